# baseline (speedup 1.0000x reference)
import jax
import jax.numpy as jnp
from jax.experimental import pallas as pl
from jax.experimental.pallas import tpu as pltpu

D_MODEL = 768
NUM_EXPERTS = 64
TOP_K = 8
N_TOKENS = 32768
BLOCK = 4096
GRID = N_TOKENS // BLOCK


def _axis0_reduce(x, op):
    while x.shape[0] > 8:
        h = x.shape[0] // 2
        x = op(x[:h], x[h:])
    return x


def _router_body(et_ref, b_ref, u_ref, ti_ref, ts_ref, s_ref, aux_ref,
                 acc, s_prev):
    step = pl.program_id(0)

    @pl.when(step == 0)
    def _init():
        acc[...] = jnp.zeros_like(acc)

    @pl.when(step > 0)
    def _topk():
        iota = jax.lax.broadcasted_iota(
            jnp.int32, (NUM_EXPERTS, BLOCK), 0
        ).astype(jnp.float32)
        work = s_prev[...]
        vals = []
        idxs = []
        for _ in range(TOP_K):
            mx = jnp.max(_axis0_reduce(work, jnp.maximum), axis=0,
                         keepdims=True)
            hit = work == mx
            idx = jnp.min(
                _axis0_reduce(jnp.where(hit, iota, 64.0), jnp.minimum),
                axis=0, keepdims=True,
            )
            vals.append(mx)
            idxs.append(idx)
            work = jnp.where(hit, -1.0, work)
        ts_ref[...] = jnp.transpose(jnp.concatenate(vals, axis=0))
        ti_ref[...] = jnp.transpose(
            jnp.concatenate(idxs, axis=0).astype(jnp.int32))

    @pl.when(step < GRID)
    def _dense():
        logits = (
            jax.lax.dot_general(
                et_ref[...], u_ref[...],
                (((1,), (1,)), ((), ())),
                preferred_element_type=jnp.float32,
            )
            + b_ref[...]
        )

        m = jnp.max(_axis0_reduce(logits, jnp.maximum), axis=0,
                    keepdims=True)
        e = jnp.exp(logits - m)
        den = jnp.sum(_axis0_reduce(e, jnp.add), axis=0, keepdims=True)
        s = e * (1.0 / den)
        s_ref[...] = jnp.transpose(s)
        s_prev[...] = s

        sp = s
        while sp.shape[1] > 128:
            h = sp.shape[1] // 2
            sp = sp[:, :h] + sp[:, h:]
        acc[...] += sp

    @pl.when(step == GRID)
    def _finish():
        mean = jnp.sum(acc[...], axis=1, keepdims=True) * (1.0 / N_TOKENS)
        aux_ref[0, 0] = jnp.sum(mean * mean) * NUM_EXPERTS


def kernel(u, E, bias):
    e_t = E.T
    bias2d = bias.reshape(NUM_EXPERTS, 1)
    last = GRID - 1
    topk_i, topk_s, scores, aux = pl.pallas_call(
        _router_body,
        grid=(GRID + 1,),
        in_specs=[
            pl.BlockSpec((NUM_EXPERTS, D_MODEL), lambda i: (0, 0)),
            pl.BlockSpec((NUM_EXPERTS, 1), lambda i: (0, 0)),
            pl.BlockSpec((BLOCK, D_MODEL),
                         lambda i: (jnp.minimum(i, last), 0)),
        ],
        out_specs=[
            pl.BlockSpec((BLOCK, TOP_K),
                         lambda i: (jnp.maximum(i - 1, 0), 0)),
            pl.BlockSpec((BLOCK, TOP_K),
                         lambda i: (jnp.maximum(i - 1, 0), 0)),
            pl.BlockSpec((BLOCK, NUM_EXPERTS),
                         lambda i: (jnp.minimum(i, last), 0)),
            pl.BlockSpec(memory_space=pltpu.SMEM),
        ],
        out_shape=[
            jax.ShapeDtypeStruct((N_TOKENS, TOP_K), jnp.int32),
            jax.ShapeDtypeStruct((N_TOKENS, TOP_K), jnp.float32),
            jax.ShapeDtypeStruct((N_TOKENS, NUM_EXPERTS), jnp.float32),
            jax.ShapeDtypeStruct((1, 1), jnp.float32),
        ],
        scratch_shapes=[
            pltpu.VMEM((NUM_EXPERTS, 128), jnp.float32),
            pltpu.VMEM((NUM_EXPERTS, BLOCK), jnp.float32),
        ],
    )(e_t, bias2d, u)
    return (topk_i, topk_s, scores, aux.reshape(()))


# speedup vs baseline: 1.0352x; 1.0352x over previous
import jax
import jax.numpy as jnp
from jax.experimental import pallas as pl
from jax.experimental.pallas import tpu as pltpu

D_MODEL = 768
NUM_EXPERTS = 64
TOP_K = 8
N_TOKENS = 32768
BLOCK = 4096
GRID = N_TOKENS // BLOCK


def _axis0_reduce(x, op):
    while x.shape[0] > 8:
        h = x.shape[0] // 2
        x = op(x[:h], x[h:])
    return x


def _router_body(et_ref, b_ref, u_ref, ti_ref, ts_ref, s_ref, aux_ref, acc):
    step = pl.program_id(0)

    @pl.when(step == 0)
    def _init():
        acc[...] = jnp.zeros_like(acc)

    logits = (
        jax.lax.dot_general(
            et_ref[...], u_ref[...],
            (((1,), (1,)), ((), ())),
            preferred_element_type=jnp.float32,
        )
        + b_ref[...]
    )

    m = jnp.max(_axis0_reduce(logits, jnp.maximum), axis=0, keepdims=True)
    e = jnp.exp(logits - m)
    den = jnp.sum(_axis0_reduce(e, jnp.add), axis=0, keepdims=True)
    s = e * (1.0 / den)
    s_ref[...] = jnp.transpose(s)

    sp = s
    while sp.shape[1] > 128:
        h = sp.shape[1] // 2
        sp = sp[:, :h] + sp[:, h:]
    acc[...] += sp

    iota = jax.lax.broadcasted_iota(
        jnp.int32, (NUM_EXPERTS, BLOCK), 0
    ).astype(jnp.float32)
    work = s
    vals = []
    idxs = []
    for _ in range(TOP_K):
        mx = jnp.max(_axis0_reduce(work, jnp.maximum), axis=0, keepdims=True)
        hit = work == mx
        idx = jnp.min(
            _axis0_reduce(jnp.where(hit, iota, 64.0), jnp.minimum),
            axis=0, keepdims=True,
        )
        vals.append(mx)
        idxs.append(idx)
        work = jnp.where(hit, -1.0, work)
    ts_ref[...] = jnp.transpose(jnp.concatenate(vals, axis=0))
    ti_ref[...] = jnp.transpose(jnp.concatenate(idxs, axis=0).astype(jnp.int32))

    @pl.when(step == GRID - 1)
    def _finish():
        mean = jnp.sum(acc[...], axis=1, keepdims=True) * (1.0 / N_TOKENS)
        aux_ref[0, 0] = jnp.sum(mean * mean) * NUM_EXPERTS


def kernel(u, E, bias):
    e_t = E.T
    bias2d = bias.reshape(NUM_EXPERTS, 1)
    topk_i, topk_s, scores, aux = pl.pallas_call(
        _router_body,
        grid=(GRID,),
        in_specs=[
            pl.BlockSpec((NUM_EXPERTS, D_MODEL), lambda i: (0, 0)),
            pl.BlockSpec((NUM_EXPERTS, 1), lambda i: (0, 0)),
            pl.BlockSpec((BLOCK, D_MODEL), lambda i: (i, 0)),
        ],
        out_specs=[
            pl.BlockSpec((BLOCK, TOP_K), lambda i: (i, 0)),
            pl.BlockSpec((BLOCK, TOP_K), lambda i: (i, 0)),
            pl.BlockSpec((BLOCK, NUM_EXPERTS), lambda i: (i, 0)),
            pl.BlockSpec(memory_space=pltpu.SMEM),
        ],
        out_shape=[
            jax.ShapeDtypeStruct((N_TOKENS, TOP_K), jnp.int32),
            jax.ShapeDtypeStruct((N_TOKENS, TOP_K), jnp.float32),
            jax.ShapeDtypeStruct((N_TOKENS, NUM_EXPERTS), jnp.float32),
            jax.ShapeDtypeStruct((1, 1), jnp.float32),
        ],
        scratch_shapes=[pltpu.VMEM((NUM_EXPERTS, 128), jnp.float32)],
    )(e_t, bias2d, u)
    return (topk_i, topk_s, scores, aux.reshape(()))
